# baseline (speedup 1.0000x reference)
import functools

import jax
import jax.numpy as jnp
from jax import lax
from jax.experimental import pallas as pl
from jax.experimental.pallas import tpu as pltpu
from jax.experimental.pallas import tpu_sc as plsc

VOCAB = 100000
EMBED_DIM = 300
BATCH = 1024
MAX_NORM = 1.0
N_TILE = 4096
NUM_TILES = -(-VOCAB // N_TILE)
VOCAB_PAD = NUM_TILES * N_TILE


def _gather_sc(slab_idx, table):
    info = plsc.get_sparse_core_info()
    nw = info.num_cores * info.num_subcores
    b_per_w = BATCH // nw
    mesh = plsc.VectorSubcoreMesh(core_axis_name="c", subcore_axis_name="s")

    @functools.partial(
        pl.kernel,
        out_type=jax.ShapeDtypeStruct((BATCH, 8, EMBED_DIM), jnp.bfloat16),
        mesh=mesh,
        scratch_types=[
            pltpu.VMEM((b_per_w,), jnp.int32),
            pltpu.VMEM((b_per_w, 8, EMBED_DIM), jnp.bfloat16),
            pltpu.SemaphoreType.DMA,
        ],
    )
    def gather_kernel(idx_hbm, table_hbm, out_hbm, idx_v, slab_v, sem):
        wid = lax.axis_index("s") * info.num_cores + lax.axis_index("c")
        base = wid * b_per_w
        pltpu.sync_copy(idx_hbm.at[pl.ds(base, b_per_w)], idx_v)
        for i in range(b_per_w):
            vec = idx_v[pl.ds((i // 16) * 16, 16)]
            start = pl.multiple_of(vec[i % 16] * 8, 8)
            pltpu.async_copy(
                table_hbm.at[pl.ds(start, 8)],
                slab_v.at[i],
                sem,
            )
        for i in range(b_per_w):
            pltpu.make_async_copy(
                table_hbm.at[pl.ds(0, 8)],
                slab_v.at[i],
                sem,
            ).wait()
        pltpu.sync_copy(slab_v, out_hbm.at[pl.ds(base, b_per_w)])

    return gather_kernel(slab_idx, table)


def _project_tc(xg, r8, W, b2):
    def mm_kernel(xg_ref, r8_ref, w_ref, b_ref, o_ref, xs_ref):
        @pl.when(pl.program_id(0) == 0)
        def _():
            r8v = r8_ref[...]
            xv = jnp.zeros((BATCH, EMBED_DIM), jnp.float32)
            for k in range(8):
                row = xg_ref[:, k, :].astype(jnp.float32)
                xv = jnp.where(r8v == k, row, xv)
            norm = jnp.sqrt(jnp.sum(xv * xv, axis=1, keepdims=True))
            scale = jnp.minimum(1.0, MAX_NORM / jnp.maximum(norm, 1e-12))
            xs_ref[...] = (xv * scale).astype(jnp.bfloat16)

        acc = lax.dot_general(
            xs_ref[...], w_ref[...], (((1,), (1,)), ((), ())),
            preferred_element_type=jnp.float32,
        )
        o_ref[...] = (acc + b_ref[0]).astype(jnp.bfloat16)

    return pl.pallas_call(
        mm_kernel,
        grid=(NUM_TILES,),
        in_specs=[
            pl.BlockSpec((BATCH, 8, EMBED_DIM), lambda j: (0, 0, 0)),
            pl.BlockSpec((BATCH, 1), lambda j: (0, 0)),
            pl.BlockSpec((N_TILE, EMBED_DIM), lambda j: (j, 0)),
            pl.BlockSpec((1, 1, N_TILE), lambda j: (j, 0, 0)),
        ],
        out_specs=pl.BlockSpec((BATCH, N_TILE), lambda j: (0, j)),
        out_shape=jax.ShapeDtypeStruct((BATCH, VOCAB), jnp.bfloat16),
        scratch_shapes=[pltpu.VMEM((BATCH, EMBED_DIM), jnp.bfloat16)],
    )(xg, r8, W, b2)


def kernel(inputs, emb_table, W, b):
    Wb = W.astype(jnp.bfloat16)
    b2 = jnp.pad(b, (0, VOCAB_PAD - VOCAB)).reshape(NUM_TILES, 1, N_TILE)
    idx = inputs.astype(jnp.int32)
    xg = _gather_sc(idx // 8, emb_table.astype(jnp.bfloat16))
    r8 = (idx % 8).reshape(BATCH, 1)
    return _project_tc(xg, r8, Wb, b2).astype(jnp.float32)


# speedup vs baseline: 1.0520x; 1.0520x over previous
import functools

import jax
import jax.numpy as jnp
from jax import lax
from jax.experimental import pallas as pl
from jax.experimental.pallas import tpu as pltpu
from jax.experimental.pallas import tpu_sc as plsc

VOCAB = 100000
EMBED_DIM = 300
BATCH = 1024
MAX_NORM = 1.0
N_TILE = 4096
NUM_TILES = -(-VOCAB // N_TILE)
VOCAB_PAD = NUM_TILES * N_TILE


def _gather_sc(idx, table):
    info = plsc.get_sparse_core_info()
    nw = info.num_cores * info.num_subcores
    b_per_w = BATCH // nw
    mesh = plsc.VectorSubcoreMesh(core_axis_name="c", subcore_axis_name="s")

    @functools.partial(
        pl.kernel,
        out_type=jax.ShapeDtypeStruct((BATCH, EMBED_DIM), jnp.float32),
        mesh=mesh,
        scratch_types=[
            pltpu.VMEM((b_per_w,), jnp.int32),
            pltpu.VMEM((b_per_w, EMBED_DIM), jnp.float32),
            pltpu.SemaphoreType.DMA,
        ],
    )
    def gather_kernel(idx_hbm, table_hbm, out_hbm, idx_v, rows_v, sem):
        wid = lax.axis_index("s") * info.num_cores + lax.axis_index("c")
        base = wid * b_per_w
        pltpu.sync_copy(idx_hbm.at[pl.ds(base, b_per_w)], idx_v)
        for i in range(b_per_w):
            vec = idx_v[pl.ds((i // 16) * 16, 16)]
            pltpu.async_copy(
                table_hbm.at[pl.ds(vec[i % 16], 1)],
                rows_v.at[pl.ds(i, 1)],
                sem,
            )
        for i in range(b_per_w):
            pltpu.make_async_copy(
                table_hbm.at[pl.ds(0, 1)],
                rows_v.at[pl.ds(i, 1)],
                sem,
            ).wait()
        pltpu.sync_copy(rows_v, out_hbm.at[pl.ds(base, b_per_w)])

    return gather_kernel(idx, table)


def _project_tc(x, W, b2):
    def mm_kernel(x_ref, w_ref, b_ref, o_ref, xs_ref):
        @pl.when(pl.program_id(0) == 0)
        def _():
            xv = x_ref[...].astype(jnp.float32)
            norm = jnp.sqrt(jnp.sum(xv * xv, axis=1, keepdims=True))
            scale = jnp.minimum(1.0, MAX_NORM / jnp.maximum(norm, 1e-12))
            xs_ref[...] = (xv * scale).astype(jnp.bfloat16)

        acc = lax.dot_general(
            xs_ref[...], w_ref[...], (((1,), (1,)), ((), ())),
            preferred_element_type=jnp.float32,
        )
        o_ref[...] = (acc + b_ref[0]).astype(jnp.bfloat16)

    return pl.pallas_call(
        mm_kernel,
        grid=(NUM_TILES,),
        in_specs=[
            pl.BlockSpec((BATCH, EMBED_DIM), lambda j: (0, 0)),
            pl.BlockSpec((N_TILE, EMBED_DIM), lambda j: (j, 0)),
            pl.BlockSpec((1, 1, N_TILE), lambda j: (j, 0, 0)),
        ],
        out_specs=pl.BlockSpec((BATCH, N_TILE), lambda j: (0, j)),
        out_shape=jax.ShapeDtypeStruct((BATCH, VOCAB), jnp.bfloat16),
        scratch_shapes=[pltpu.VMEM((BATCH, EMBED_DIM), jnp.bfloat16)],
    )(x, W, b2)


def kernel(inputs, emb_table, W, b):
    Wb = W.astype(jnp.bfloat16)
    b2 = jnp.pad(b, (0, VOCAB_PAD - VOCAB)).reshape(NUM_TILES, 1, N_TILE)
    x = _gather_sc(inputs.astype(jnp.int32), emb_table)
    return _project_tc(x, Wb, b2).astype(jnp.float32)
